# baseline (speedup 1.0000x reference)
import functools
import math

import jax
import jax.numpy as jnp
from jax import lax
from jax.experimental import pallas as pl
from jax.experimental.pallas import tpu as pltpu
from jax.experimental.pallas import tpu_sc as plsc

_NUM_CODES = 1000000
_CODE_DIM = 16
_EMBED_DIM = 32
_HIDDEN = 128
_OUT = 64
_B = 4096
_T = 20

_N_IDX = _B * _T * _CODE_DIM
_ROWS = _B * _T
_FEAT = _CODE_DIM * _EMBED_DIM
_NPART = 4
_XROWS = _N_IDX // 4

_INFO = plsc.get_sparse_core_info()
_NC = _INFO.num_cores
_NS = _INFO.num_subcores
_NW = _NC * _NS
_PER_W = _N_IDX // _NW
_CHUNK = 1024
_STREAMS = _CHUNK // 128
_OUTER = _PER_W // _CHUNK


def _sc_gather(codes2d, table):
    mesh = plsc.VectorSubcoreMesh(core_axis_name="c", subcore_axis_name="s")

    @functools.partial(
        pl.kernel,
        mesh=mesh,
        out_type=jax.ShapeDtypeStruct((_XROWS, 128), jnp.float32),
        scratch_types=[
            pltpu.VMEM((_STREAMS, 128), jnp.int32),
            pltpu.VMEM((_CHUNK, _EMBED_DIM), jnp.float32),
            pltpu.VMEM((_CHUNK // 4, 128), jnp.float32),
            pltpu.SemaphoreType.DMA,
        ],
        compiler_params=pltpu.CompilerParams(use_tc_tiling_on_sc=False),
    )
    def k(codes_hbm, table_hbm, out_hbm, idx_v, rows_v, rows2_v, sem):
        wid = lax.axis_index("s") * _NC + lax.axis_index("c")

        def body(outer, carry):
            row0 = wid * (_OUTER * _STREAMS) + outer * _STREAMS
            xr0 = (wid * _PER_W + outer * _CHUNK) // 4
            pltpu.sync_copy(codes_hbm.at[pl.ds(row0, _STREAMS)], idx_v)
            cps = []
            for j in range(_STREAMS):
                cp = pltpu.async_copy(
                    table_hbm.at[idx_v.at[j]],
                    rows_v.at[pl.ds(j * 128, 128)],
                    sem,
                )
                cps.append(cp)
            for cp in cps:
                cp.wait()

            def rbody(i, c):
                for half in range(2):
                    for src in range(4):
                        rows2_v[i, pl.ds(src * 32 + half * 16, 16)] = (
                            rows_v[i * 4 + src, pl.ds(half * 16, 16)]
                        )
                return c

            lax.fori_loop(0, _CHUNK // 4, rbody, 0)
            pltpu.sync_copy(
                rows2_v,
                out_hbm.at[pl.ds(xr0, _CHUNK // 4)],
            )
            return carry

        lax.fori_loop(0, _OUTER, body, 0)

    return k(codes2d, table)


_ROW_BLK = 1024
_PBLK = _ROWS // _ROW_BLK


def _mlp_body(x0_ref, x1_ref, x2_ref, x3_ref, w1_ref, b1_ref, gamma_ref,
              beta_ref, w2_ref, b2_ref, o_ref):
    h = b1_ref[...]
    for q, xq_ref in enumerate((x0_ref, x1_ref, x2_ref, x3_ref)):
        h = h + jnp.dot(
            xq_ref[...],
            w1_ref[pl.ds(q * 128, 128), :],
            preferred_element_type=jnp.float32,
        )
    h = 0.5 * h * (1.0 + lax.erf(h * (1.0 / math.sqrt(2.0))))
    mu = jnp.mean(h, axis=-1, keepdims=True)
    var = jnp.mean((h - mu) ** 2, axis=-1, keepdims=True)
    h = (h - mu) * lax.rsqrt(var + 1e-5)
    h = h * gamma_ref[...] + beta_ref[...]
    o_ref[...] = jnp.dot(h, w2_ref[...], preferred_element_type=jnp.float32) + b2_ref[...]


def _part_spec(q):
    return pl.BlockSpec((_ROW_BLK, 128), lambda i, q=q: (q * _PBLK + i, 0))


def _tc_mlp(x128, W1, b1, gamma, beta, W2, b2):
    grid = (_PBLK,)
    return pl.pallas_call(
        _mlp_body,
        grid=grid,
        in_specs=[
            _part_spec(0),
            _part_spec(1),
            _part_spec(2),
            _part_spec(3),
            pl.BlockSpec((_FEAT, _HIDDEN), lambda i: (0, 0)),
            pl.BlockSpec((1, _HIDDEN), lambda i: (0, 0)),
            pl.BlockSpec((1, _HIDDEN), lambda i: (0, 0)),
            pl.BlockSpec((1, _HIDDEN), lambda i: (0, 0)),
            pl.BlockSpec((_HIDDEN, _OUT), lambda i: (0, 0)),
            pl.BlockSpec((1, _OUT), lambda i: (0, 0)),
        ],
        out_specs=pl.BlockSpec((_ROW_BLK, _OUT), lambda i: (i, 0)),
        out_shape=jax.ShapeDtypeStruct((_ROWS, _OUT), jnp.float32),
    )(x128, x128, x128, x128, W1, b1, gamma, beta, W2, b2)


def kernel(codes, table, W1, b1, gamma, beta, W2, b2):
    codes_pm = (
        codes.reshape(_ROWS, _NPART, 4)
        .transpose(1, 0, 2)
        .reshape(_N_IDX // 128, 128)
    )
    x128 = _sc_gather(codes_pm, table)
    out2d = _tc_mlp(
        x128,
        W1,
        b1.reshape(1, _HIDDEN),
        gamma.reshape(1, _HIDDEN),
        beta.reshape(1, _HIDDEN),
        W2,
        b2.reshape(1, _OUT),
    )
    return out2d.reshape(_B, _T, _OUT)
